# Initial kernel scaffold; baseline (speedup 1.0000x reference)
import jax
import jax.numpy as jnp
from jax.experimental import pallas as pl


def kernel(x, target):
    raise NotImplementedError("write your pallas kernel here")


# speedup vs baseline: 1.7420x; 1.7420x over previous
import math

import jax
import jax.numpy as jnp
from jax.experimental import pallas as pl
from jax.experimental.pallas import tpu as pltpu

_V = 100000
_B = 1024
_SMOOTH = 0.1
_CONF = 1.0 - _SMOOTH
_EPS = _SMOOTH / (_V - 1)
_CONST = _B * ((_V - 1) * _EPS * math.log(_EPS) + _CONF * math.log(_CONF))

_ROWS = 16


def _body(t_ref, x_ref, o_ref):
    i = pl.program_id(0)
    x = x_ref[...]
    t = t_ref[...]
    cols = jax.lax.broadcasted_iota(jnp.int32, x.shape, 1)
    g = jnp.sum(jnp.where(cols == t, x, jnp.float32(0.0)))
    s = jnp.sum(x)
    part = jnp.float32(_EPS) * s + jnp.float32(_CONF - _EPS) * g

    @pl.when(i == 0)
    def _():
        o_ref[0, 0] = jnp.float32(_CONST)

    o_ref[0, 0] = o_ref[0, 0] - part


def kernel(x, target):
    t2 = target.astype(jnp.int32).reshape(_B, 1)
    out = pl.pallas_call(
        _body,
        grid=(_B // _ROWS,),
        in_specs=[
            pl.BlockSpec((_ROWS, 1), lambda i: (i, 0)),
            pl.BlockSpec((_ROWS, _V), lambda i: (i, 0)),
        ],
        out_specs=pl.BlockSpec(memory_space=pltpu.SMEM),
        out_shape=jax.ShapeDtypeStruct((1, 1), jnp.float32),
    )(t2, x)
    return out[0, 0]
